# baseline (speedup 1.0000x reference)
import functools

import jax
import jax.numpy as jnp
from jax import lax
from jax.experimental import pallas as pl
from jax.experimental.pallas import tpu as pltpu
from jax.experimental.pallas import tpu_sc as plsc

N = 10000
E = 320000
D = 128
G = 64
NCLS = 10

NC = 2
NS = 16
NW = NC * NS
CH = 80
NBUF = 4
GK = 5
NGRP = 25
NK = GK * NGRP
EPT = E // NW
NPT = 624
EXB = NS * NPT
EXN = N - EXB

def _agg_body(h_hbm, seed_hbm, src_hbm, dst_hbm, out_hbm,
              src_v, dst_v, rows_v, acc, gsem, ssem, sisem, disem):
    c = lax.axis_index("c")
    s = lax.axis_index("s")
    w = c * NS + s
    pltpu.async_copy(src_hbm.at[w, 0], src_v.at[0], sisem.at[0])
    pltpu.async_copy(dst_hbm.at[w, 0], dst_v.at[0], disem.at[0])
    row0 = s * NPT

    @pl.when(c == 0)
    def _():
        pltpu.sync_copy(h_hbm.at[pl.ds(row0, NPT)], acc.at[pl.ds(row0, NPT)])

        @pl.when(s == NS - 1)
        def _():
            pltpu.sync_copy(h_hbm.at[pl.ds(EXB, EXN)],
                            acc.at[pl.ds(EXB, EXN)])

    @pl.when(c != 0)
    def _():
        pltpu.sync_copy(seed_hbm.at[pl.ds(row0, NPT)],
                        acc.at[pl.ds(row0, NPT)])

        @pl.when(s == NS - 1)
        def _():
            pltpu.sync_copy(seed_hbm.at[pl.ds(EXB, EXN)],
                            acc.at[pl.ds(EXB, EXN)])

    plsc.subcore_barrier()

    def wait_gather(bslot):
        pltpu.make_async_copy(h_hbm.at[src_v.at[0, 0]], rows_v.at[bslot],
                              gsem.at[bslot]).wait()

    def wait_scatter(bslot):
        pltpu.make_async_copy(rows_v.at[bslot], acc.at[dst_v.at[0, 0]],
                              ssem.at[bslot]).wait()

    def wait_stage(ring_hbm, ring_v, sem, slot):
        pltpu.make_async_copy(ring_hbm.at[w, 0], ring_v.at[slot],
                              sem.at[slot]).wait()

    wait_stage(src_hbm, src_v, sisem, 0)
    pltpu.async_copy(h_hbm.at[src_v.at[0, 0]], rows_v.at[0], gsem.at[0])

    def body(j, carry):
        g = j // GK
        bpos = j % GK
        bb = j % NBUF
        gs = g % 2

        @pl.when(bpos == 0)
        def _():
            wait_stage(dst_hbm, dst_v, disem, gs)

        @pl.when((bpos == 2) & (g < NGRP - 1))
        def _():
            pltpu.async_copy(src_hbm.at[w, g + 1],
                             src_v.at[1 - gs], sisem.at[1 - gs])
            pltpu.async_copy(dst_hbm.at[w, g + 1],
                             dst_v.at[1 - gs], disem.at[1 - gs])

        @pl.when((bpos == GK - 1) & (j + 1 < NK))
        def _():
            wait_stage(src_hbm, src_v, sisem, 1 - gs)

        bb1 = (j + 1) % NBUF

        @pl.when(j + 1 < NK)
        def _():
            @pl.when(j >= NBUF - 1)
            def _():
                wait_scatter(bb1)
            gnext = (j + 1) // GK
            pltpu.async_copy(
                h_hbm.at[src_v.at[gnext % 2, (j + 1) % GK]],
                rows_v.at[bb1], gsem.at[bb1])

        wait_gather(bb)
        pltpu.async_copy(rows_v.at[bb], acc.at[dst_v.at[gs, bpos]],
                         ssem.at[bb], add=True)
        return carry

    lax.fori_loop(0, NK, body, 0)
    for jj in range(NK - NBUF, NK):
        wait_scatter(jj % NBUF)
    plsc.subcore_barrier()
    pltpu.sync_copy(acc.at[pl.ds(row0, NPT)],
                    out_hbm.at[c, pl.ds(row0, NPT)])

    @pl.when(s == NS - 1)
    def _():
        pltpu.sync_copy(acc.at[pl.ds(EXB, EXN)],
                        out_hbm.at[c, pl.ds(EXB, EXN)])


@functools.cache
def _make_agg():
    mesh = plsc.VectorSubcoreMesh(
        core_axis_name="c", subcore_axis_name="s",
        num_cores=NC, num_subcores=NS)
    return pl.kernel(
        _agg_body,
        out_type=jax.ShapeDtypeStruct((NC, N, D), jnp.float32),
        mesh=mesh,
        scratch_types=[
            pltpu.VMEM((2, GK, CH), jnp.int32),
            pltpu.VMEM((2, GK, CH), jnp.int32),
            pltpu.VMEM((NBUF, CH, D), jnp.float32),
            pltpu.VMEM_SHARED((N, D), jnp.float32),
            pltpu.SemaphoreType.DMA((NBUF,)),
            pltpu.SemaphoreType.DMA((NBUF,)),
            pltpu.SemaphoreType.DMA((2,)),
            pltpu.SemaphoreType.DMA((2,)),
        ],
    )


_BLK = 2000
_NBLK = N // _BLK


def _mlp1_body(agg_ref, w1_ref, b1_ref, y_ref, st_ref):
    i = pl.program_id(0)
    hin = agg_ref[0] + agg_ref[1]
    y = jnp.dot(hin, w1_ref[...], preferred_element_type=jnp.float32)
    y = y + b1_ref[...]
    y_ref[...] = y
    cs = jnp.sum(y, axis=0, keepdims=True)
    cq = jnp.sum(y * y, axis=0, keepdims=True)
    upd = jnp.concatenate(
        [cs, cq, jnp.zeros((6, D), jnp.float32)], axis=0)

    @pl.when(i == 0)
    def _():
        st_ref[...] = jnp.zeros_like(st_ref)

    st_ref[...] += upd


_mlp1 = pl.pallas_call(
    _mlp1_body,
    grid=(_NBLK,),
    in_specs=[
        pl.BlockSpec((NC, _BLK, D), lambda i: (0, i, 0)),
        pl.BlockSpec((D, D), lambda i: (0, 0)),
        pl.BlockSpec((1, D), lambda i: (0, 0)),
    ],
    out_specs=[
        pl.BlockSpec((_BLK, D), lambda i: (i, 0)),
        pl.BlockSpec((8, D), lambda i: (0, 0)),
    ],
    out_shape=[
        jax.ShapeDtypeStruct((N, D), jnp.float32),
        jax.ShapeDtypeStruct((8, D), jnp.float32),
    ],
)


def _mlp2_body(y_ref, a_ref, c_ref, w2_ref, b2_ref, z_ref):
    t = jnp.maximum(y_ref[...] * a_ref[...] + c_ref[...], 0.0)
    z = jnp.dot(t, w2_ref[...], preferred_element_type=jnp.float32)
    z_ref[...] = jnp.maximum(z + b2_ref[...], 0.0)


_mlp2 = pl.pallas_call(
    _mlp2_body,
    grid=(_NBLK,),
    in_specs=[
        pl.BlockSpec((_BLK, D), lambda i: (i, 0)),
        pl.BlockSpec((1, D), lambda i: (0, 0)),
        pl.BlockSpec((1, D), lambda i: (0, 0)),
        pl.BlockSpec((D, D), lambda i: (0, 0)),
        pl.BlockSpec((1, D), lambda i: (0, 0)),
    ],
    out_specs=pl.BlockSpec((_BLK, D), lambda i: (i, 0)),
    out_shape=jax.ShapeDtypeStruct((N, D), jnp.float32),
)


def _pool_head_body(h_ref, b_ref, w1_ref, b1_ref, w2_ref, b2_ref,
                    out_ref, acc_ref):
    i = pl.program_id(0)

    @pl.when(i == 0)
    def _():
        acc_ref[...] = jnp.zeros_like(acc_ref)

    seg = b_ref[0, 0, :]
    gid = lax.broadcasted_iota(jnp.int32, (G, _BLK), 0)
    onehot = (gid == seg[None, :]).astype(jnp.float32)
    acc_ref[...] += jnp.dot(onehot, h_ref[...],
                            preferred_element_type=jnp.float32)

    @pl.when(i == _NBLK - 1)
    def _():
        p = acc_ref[...]
        r = jnp.maximum(
            jnp.dot(p, w1_ref[...], preferred_element_type=jnp.float32)
            + b1_ref[...], 0.0)
        o = jnp.dot(r, w2_ref[...], preferred_element_type=jnp.float32)
        o = o + b2_ref[...]
        col = lax.broadcasted_iota(jnp.int32, (G, D), 1)
        valid = col < NCLS
        om = jnp.where(valid, o, -jnp.inf)
        m = jnp.max(om, axis=1, keepdims=True)
        e = jnp.where(valid, jnp.exp(om - m), 0.0)
        lse = jnp.log(jnp.sum(e, axis=1, keepdims=True)) + m
        out_ref[...] = o - lse


_pool_head = pl.pallas_call(
    _pool_head_body,
    grid=(_NBLK,),
    in_specs=[
        pl.BlockSpec((_BLK, D), lambda i: (i, 0)),
        pl.BlockSpec((1, 1, _BLK), lambda i: (i, 0, 0)),
        pl.BlockSpec((D, D), lambda i: (0, 0)),
        pl.BlockSpec((1, D), lambda i: (0, 0)),
        pl.BlockSpec((D, D), lambda i: (0, 0)),
        pl.BlockSpec((1, D), lambda i: (0, 0)),
    ],
    out_specs=pl.BlockSpec((G, D), lambda i: (0, 0)),
    out_shape=jax.ShapeDtypeStruct((G, D), jnp.float32),
    scratch_shapes=[pltpu.VMEM((G, D), jnp.float32)],
)


def kernel(x, edge_index, batch, params):
    srcp = edge_index[0].reshape(NW, NGRP, GK, CH)
    dstp = edge_index[1].reshape(NW, NGRP, GK, CH)
    zeros = jnp.zeros((N, D), jnp.float32)

    h = x
    for i in range(1, 6):
        p = params['conv' + str(i)]
        parts = _make_agg()(h, zeros, srcp, dstp)
        y, st = _mlp1(parts, p['W1'], p['b1'].reshape(1, D))
        mean = st[0] / N
        var = st[1] / N - mean * mean
        a = p['gamma'] * lax.rsqrt(var + 1e-5)
        cvec = p['beta'] - mean * a
        h = _mlp2(y, a.reshape(1, D), cvec.reshape(1, D),
                  p['W2'], p['b2'].reshape(1, D))

    w2h = jnp.zeros((D, D), jnp.float32).at[:, :NCLS].set(params['lin2_W'])
    b2h = jnp.zeros((1, D), jnp.float32).at[0, :NCLS].set(params['lin2_b'])
    out = _pool_head(h, batch.reshape(_NBLK, 1, _BLK).astype(jnp.int32),
                     params['lin1_W'], params['lin1_b'].reshape(1, D),
                     w2h, b2h)
    return out[:, :NCLS]


# speedup vs baseline: 1.1424x; 1.1424x over previous
import functools

import jax
import jax.numpy as jnp
from jax import lax
from jax.experimental import pallas as pl
from jax.experimental.pallas import tpu as pltpu
from jax.experimental.pallas import tpu_sc as plsc

N = 10000
E = 320000
D = 128
G = 64
NCLS = 10

NC = 2
NS = 16
NW = NC * NS
CH = 100
NBUF = 3
GK = 10
NGRP = 10
NK = GK * NGRP
EPT = E // NW
NPT = 624
EXB = NS * NPT
EXN = N - EXB

def _agg_body(h_hbm, seed_hbm, src_hbm, dst_hbm, out_hbm,
              src_v, dst_v, rows_v, acc, gsem, ssem, sisem, disem):
    c = lax.axis_index("c")
    s = lax.axis_index("s")
    w = c * NS + s
    pltpu.async_copy(src_hbm.at[w, 0], src_v.at[0], sisem.at[0])
    pltpu.async_copy(dst_hbm.at[w, 0], dst_v.at[0], disem.at[0])
    row0 = s * NPT

    @pl.when(c == 0)
    def _():
        pltpu.sync_copy(h_hbm.at[pl.ds(row0, NPT)], acc.at[pl.ds(row0, NPT)])

        @pl.when(s == NS - 1)
        def _():
            pltpu.sync_copy(h_hbm.at[pl.ds(EXB, EXN)],
                            acc.at[pl.ds(EXB, EXN)])

    @pl.when(c != 0)
    def _():
        pltpu.sync_copy(seed_hbm.at[pl.ds(row0, NPT)],
                        acc.at[pl.ds(row0, NPT)])

        @pl.when(s == NS - 1)
        def _():
            pltpu.sync_copy(seed_hbm.at[pl.ds(EXB, EXN)],
                            acc.at[pl.ds(EXB, EXN)])

    plsc.subcore_barrier()

    def wait_gather(bslot):
        pltpu.make_async_copy(h_hbm.at[src_v.at[0, 0]], rows_v.at[bslot],
                              gsem.at[bslot]).wait()

    def wait_scatter(bslot):
        pltpu.make_async_copy(rows_v.at[bslot], acc.at[dst_v.at[0, 0]],
                              ssem.at[bslot]).wait()

    def wait_stage(ring_hbm, ring_v, sem, slot):
        pltpu.make_async_copy(ring_hbm.at[w, 0], ring_v.at[slot],
                              sem.at[slot]).wait()

    wait_stage(src_hbm, src_v, sisem, 0)
    pltpu.async_copy(h_hbm.at[src_v.at[0, 0]], rows_v.at[0], gsem.at[0])

    def body(j, carry):
        g = j // GK
        bpos = j % GK
        bb = j % NBUF
        gs = g % 2

        @pl.when(bpos == 0)
        def _():
            wait_stage(dst_hbm, dst_v, disem, gs)

        @pl.when((bpos == 2) & (g < NGRP - 1))
        def _():
            pltpu.async_copy(src_hbm.at[w, g + 1],
                             src_v.at[1 - gs], sisem.at[1 - gs])
            pltpu.async_copy(dst_hbm.at[w, g + 1],
                             dst_v.at[1 - gs], disem.at[1 - gs])

        @pl.when((bpos == GK - 1) & (j + 1 < NK))
        def _():
            wait_stage(src_hbm, src_v, sisem, 1 - gs)

        bb1 = (j + 1) % NBUF

        @pl.when(j + 1 < NK)
        def _():
            @pl.when(j >= NBUF - 1)
            def _():
                wait_scatter(bb1)
            gnext = (j + 1) // GK
            pltpu.async_copy(
                h_hbm.at[src_v.at[gnext % 2, (j + 1) % GK]],
                rows_v.at[bb1], gsem.at[bb1])

        wait_gather(bb)
        pltpu.async_copy(rows_v.at[bb], acc.at[dst_v.at[gs, bpos]],
                         ssem.at[bb], add=True)
        return carry

    lax.fori_loop(0, NK, body, 0)
    for jj in range(NK - NBUF, NK):
        wait_scatter(jj % NBUF)
    plsc.subcore_barrier()
    pltpu.sync_copy(acc.at[pl.ds(row0, NPT)],
                    out_hbm.at[c, pl.ds(row0, NPT)])

    @pl.when(s == NS - 1)
    def _():
        pltpu.sync_copy(acc.at[pl.ds(EXB, EXN)],
                        out_hbm.at[c, pl.ds(EXB, EXN)])


@functools.cache
def _make_agg():
    mesh = plsc.VectorSubcoreMesh(
        core_axis_name="c", subcore_axis_name="s",
        num_cores=NC, num_subcores=NS)
    return pl.kernel(
        _agg_body,
        out_type=jax.ShapeDtypeStruct((NC, N, D), jnp.float32),
        mesh=mesh,
        scratch_types=[
            pltpu.VMEM((2, GK, CH), jnp.int32),
            pltpu.VMEM((2, GK, CH), jnp.int32),
            pltpu.VMEM((NBUF, CH, D), jnp.float32),
            pltpu.VMEM_SHARED((N, D), jnp.float32),
            pltpu.SemaphoreType.DMA((NBUF,)),
            pltpu.SemaphoreType.DMA((NBUF,)),
            pltpu.SemaphoreType.DMA((2,)),
            pltpu.SemaphoreType.DMA((2,)),
        ],
    )


_BLK = 2000
_NBLK = N // _BLK


def _mlp1_body(agg_ref, w1_ref, b1_ref, y_ref, st_ref):
    i = pl.program_id(0)
    hin = agg_ref[0] + agg_ref[1]
    y = jnp.dot(hin, w1_ref[...], preferred_element_type=jnp.float32)
    y = y + b1_ref[...]
    y_ref[...] = y
    cs = jnp.sum(y, axis=0, keepdims=True)
    cq = jnp.sum(y * y, axis=0, keepdims=True)
    upd = jnp.concatenate(
        [cs, cq, jnp.zeros((6, D), jnp.float32)], axis=0)

    @pl.when(i == 0)
    def _():
        st_ref[...] = jnp.zeros_like(st_ref)

    st_ref[...] += upd


_mlp1 = pl.pallas_call(
    _mlp1_body,
    grid=(_NBLK,),
    in_specs=[
        pl.BlockSpec((NC, _BLK, D), lambda i: (0, i, 0)),
        pl.BlockSpec((D, D), lambda i: (0, 0)),
        pl.BlockSpec((1, D), lambda i: (0, 0)),
    ],
    out_specs=[
        pl.BlockSpec((_BLK, D), lambda i: (i, 0)),
        pl.BlockSpec((8, D), lambda i: (0, 0)),
    ],
    out_shape=[
        jax.ShapeDtypeStruct((N, D), jnp.float32),
        jax.ShapeDtypeStruct((8, D), jnp.float32),
    ],
)


def _mlp2_body(y_ref, a_ref, c_ref, w2_ref, b2_ref, z_ref):
    t = jnp.maximum(y_ref[...] * a_ref[...] + c_ref[...], 0.0)
    z = jnp.dot(t, w2_ref[...], preferred_element_type=jnp.float32)
    z_ref[...] = jnp.maximum(z + b2_ref[...], 0.0)


_mlp2 = pl.pallas_call(
    _mlp2_body,
    grid=(_NBLK,),
    in_specs=[
        pl.BlockSpec((_BLK, D), lambda i: (i, 0)),
        pl.BlockSpec((1, D), lambda i: (0, 0)),
        pl.BlockSpec((1, D), lambda i: (0, 0)),
        pl.BlockSpec((D, D), lambda i: (0, 0)),
        pl.BlockSpec((1, D), lambda i: (0, 0)),
    ],
    out_specs=pl.BlockSpec((_BLK, D), lambda i: (i, 0)),
    out_shape=jax.ShapeDtypeStruct((N, D), jnp.float32),
)


def _pool_head_body(h_ref, b_ref, w1_ref, b1_ref, w2_ref, b2_ref,
                    out_ref, acc_ref):
    i = pl.program_id(0)

    @pl.when(i == 0)
    def _():
        acc_ref[...] = jnp.zeros_like(acc_ref)

    seg = b_ref[0, 0, :]
    gid = lax.broadcasted_iota(jnp.int32, (G, _BLK), 0)
    onehot = (gid == seg[None, :]).astype(jnp.float32)
    acc_ref[...] += jnp.dot(onehot, h_ref[...],
                            preferred_element_type=jnp.float32)

    @pl.when(i == _NBLK - 1)
    def _():
        p = acc_ref[...]
        r = jnp.maximum(
            jnp.dot(p, w1_ref[...], preferred_element_type=jnp.float32)
            + b1_ref[...], 0.0)
        o = jnp.dot(r, w2_ref[...], preferred_element_type=jnp.float32)
        o = o + b2_ref[...]
        col = lax.broadcasted_iota(jnp.int32, (G, D), 1)
        valid = col < NCLS
        om = jnp.where(valid, o, -jnp.inf)
        m = jnp.max(om, axis=1, keepdims=True)
        e = jnp.where(valid, jnp.exp(om - m), 0.0)
        lse = jnp.log(jnp.sum(e, axis=1, keepdims=True)) + m
        out_ref[...] = o - lse


_pool_head = pl.pallas_call(
    _pool_head_body,
    grid=(_NBLK,),
    in_specs=[
        pl.BlockSpec((_BLK, D), lambda i: (i, 0)),
        pl.BlockSpec((1, 1, _BLK), lambda i: (i, 0, 0)),
        pl.BlockSpec((D, D), lambda i: (0, 0)),
        pl.BlockSpec((1, D), lambda i: (0, 0)),
        pl.BlockSpec((D, D), lambda i: (0, 0)),
        pl.BlockSpec((1, D), lambda i: (0, 0)),
    ],
    out_specs=pl.BlockSpec((G, D), lambda i: (0, 0)),
    out_shape=jax.ShapeDtypeStruct((G, D), jnp.float32),
    scratch_shapes=[pltpu.VMEM((G, D), jnp.float32)],
)


def kernel(x, edge_index, batch, params):
    srcp = edge_index[0].reshape(NW, NGRP, GK, CH)
    dstp = edge_index[1].reshape(NW, NGRP, GK, CH)
    zeros = jnp.zeros((N, D), jnp.float32)

    h = x
    for i in range(1, 6):
        p = params['conv' + str(i)]
        parts = _make_agg()(h, zeros, srcp, dstp)
        y, st = _mlp1(parts, p['W1'], p['b1'].reshape(1, D))
        mean = st[0] / N
        var = st[1] / N - mean * mean
        a = p['gamma'] * lax.rsqrt(var + 1e-5)
        cvec = p['beta'] - mean * a
        h = _mlp2(y, a.reshape(1, D), cvec.reshape(1, D),
                  p['W2'], p['b2'].reshape(1, D))

    w2h = jnp.zeros((D, D), jnp.float32).at[:, :NCLS].set(params['lin2_W'])
    b2h = jnp.zeros((1, D), jnp.float32).at[0, :NCLS].set(params['lin2_b'])
    out = _pool_head(h, batch.reshape(_NBLK, 1, _BLK).astype(jnp.int32),
                     params['lin1_W'], params['lin1_b'].reshape(1, D),
                     w2h, b2h)
    return out[:, :NCLS]
